# baseline (speedup 1.0000x reference)
import functools

import jax
import jax.numpy as jnp
from jax import lax
from jax.experimental import pallas as pl
from jax.experimental.pallas import tpu as pltpu
from jax.experimental.pallas import tpu_sc as plsc

NUM_FEATURES = 128
NUM_SAMPLES = 100000
NUM_CLASSES = 1000
TEMP = 0.05
BATCH = 1024

NC = 2
NS = 16
NW = NC * NS
CHM = 128
KM = 24
MAIN = NW * KM * CHM
TAILF = 13
REM = 32
REM_OFF = MAIN + TAILF * CHM
ACC = 1024
ROWS = ACC // NS
TGT = BATCH // NW
NBUF = 4


def _sc_body(feat_hbm, labflat_hbm, idx_hbm, zs_hbm, zc_hbm, ones_hbm,
             sums_hbm, counts_hbm, targets_hbm,
             lab_v, labt_v, labr_v, feat_a, feat_b, feat_c, feat_d,
             featt_v, ones_v, idx_v, tgt_v, s_acc, c_acc,
             sem, lsem, tsem, sema, semb, semc, semd,
             ssema, ssemb, ssemc, ssemd, csem, tssem):
    cid = lax.axis_index("c")
    sid = lax.axis_index("s")
    wid = cid * NS + sid

    bufs = (feat_a, feat_b, feat_c, feat_d)
    dsems = (sema, semb, semc, semd)
    ssems = (ssema, ssemb, ssemc, ssemd)

    def feat_src(k):
        base = pl.multiple_of((wid * KM + k) * CHM, 8)
        return feat_hbm.at[pl.ds(base, CHM)]

    def lab_src(k):
        base = pl.multiple_of((wid * KM + k) * CHM, 8)
        return labflat_hbm.at[pl.ds(base, CHM)]

    def lab_fire(k, carry):
        pltpu.async_copy(lab_src(k), lab_v.at[k], lsem)
        return carry

    lax.fori_loop(0, KM, lab_fire, 0)
    for r in range(NBUF):
        pltpu.async_copy(feat_src(r), bufs[r], dsems[r])

    tmain = pl.multiple_of(MAIN + wid * CHM, 8)

    @pl.when(wid < TAILF)
    def _():
        pltpu.async_copy(labflat_hbm.at[pl.ds(tmain, CHM)], labt_v.at[0],
                         lsem)
        pltpu.async_copy(feat_hbm.at[pl.ds(tmain, CHM)], featt_v, tsem)

    @pl.when(wid == TAILF)
    def _():
        pltpu.async_copy(labflat_hbm.at[pl.ds(REM_OFF, REM)], labr_v.at[0],
                         lsem)
        pltpu.async_copy(feat_hbm.at[pl.ds(REM_OFF, REM)],
                         featt_v.at[pl.ds(0, REM)], tsem)

    r0 = sid * ROWS
    pltpu.sync_copy(zs_hbm.at[pl.ds(r0, ROWS)], s_acc.at[pl.ds(r0, ROWS)])
    pltpu.sync_copy(zc_hbm.at[pl.ds(r0, ROWS)], c_acc.at[pl.ds(r0, ROWS)])
    pltpu.sync_copy(ones_hbm, ones_v)

    tbase = pl.multiple_of(wid * TGT, 8)
    pltpu.sync_copy(idx_hbm.at[pl.ds(tbase, TGT)], idx_v)
    pltpu.async_copy(labflat_hbm.at[idx_v], tgt_v, sem).wait()
    pltpu.sync_copy(tgt_v, targets_hbm.at[pl.ds(tbase, TGT)])

    def lab_drain(k, carry):
        pltpu.make_async_copy(lab_src(k), lab_v.at[k], lsem).wait()
        return carry

    lax.fori_loop(0, KM, lab_drain, 0)

    @pl.when(wid < TAILF)
    def _():
        pltpu.make_async_copy(labflat_hbm.at[pl.ds(tmain, CHM)],
                              labt_v.at[0], lsem).wait()

    @pl.when(wid == TAILF)
    def _():
        pltpu.make_async_copy(labflat_hbm.at[pl.ds(REM_OFF, REM)],
                              labr_v.at[0], lsem).wait()

    plsc.subcore_barrier()

    @pl.when(wid < TAILF)
    def _():
        pltpu.make_async_copy(feat_hbm.at[pl.ds(tmain, CHM)], featt_v,
                              tsem).wait()
        pltpu.async_copy(featt_v, s_acc.at[labt_v.at[0]], tssem, add=True)
        pltpu.async_copy(ones_v, c_acc.at[labt_v.at[0]], csem, add=True)

    @pl.when(wid == TAILF)
    def _():
        pltpu.make_async_copy(feat_hbm.at[pl.ds(REM_OFF, REM)],
                              featt_v.at[pl.ds(0, REM)], tsem).wait()
        pltpu.async_copy(featt_v.at[pl.ds(0, REM)],
                         s_acc.at[labr_v.at[0]], tssem, add=True)
        pltpu.async_copy(ones_v.at[pl.ds(0, REM)],
                         c_acc.at[labr_v.at[0]], csem, add=True)

    def feat_scatter_start(k, r):
        pltpu.async_copy(bufs[r], s_acc.at[lab_v.at[k]], ssems[r], add=True)

    def feat_scatter_wait(k, r):
        pltpu.make_async_copy(bufs[r], s_acc.at[lab_v.at[k]],
                              ssems[r]).wait()

    def cnt_scatter_start(k):
        pltpu.async_copy(ones_v, c_acc.at[lab_v.at[k]], csem, add=True)

    def cnt_scatter_wait(k, carry):
        pltpu.make_async_copy(ones_v, c_acc.at[lab_v.at[k]], csem).wait()
        return carry

    def body(i, carry):
        k0 = i * NBUF
        for r in range(NBUF):
            pltpu.make_async_copy(feat_src(k0 + r), bufs[r],
                                  dsems[r]).wait()
            feat_scatter_start(k0 + r, r)
            cnt_scatter_start(k0 + r)
        for r in range(NBUF):
            @pl.when(k0 + NBUF + r < KM)
            def _():
                feat_scatter_wait(k0 + r, r)
                pltpu.async_copy(feat_src(k0 + NBUF + r), bufs[r],
                                 dsems[r])
        return carry

    lax.fori_loop(0, KM // NBUF, body, 0)

    for r in range(NBUF):
        feat_scatter_wait(KM - NBUF + r, r)

    @pl.when(wid < TAILF)
    def _():
        pltpu.make_async_copy(featt_v, s_acc.at[labt_v.at[0]],
                              tssem).wait()
        pltpu.make_async_copy(ones_v, c_acc.at[labt_v.at[0]], csem).wait()

    @pl.when(wid == TAILF)
    def _():
        pltpu.make_async_copy(featt_v.at[pl.ds(0, REM)],
                              s_acc.at[labr_v.at[0]], tssem).wait()
        pltpu.make_async_copy(ones_v.at[pl.ds(0, REM)],
                              c_acc.at[labr_v.at[0]], csem).wait()

    lax.fori_loop(0, KM, cnt_scatter_wait, 0)

    plsc.subcore_barrier()

    pltpu.sync_copy(s_acc.at[pl.ds(r0, ROWS)],
                    sums_hbm.at[cid, pl.ds(r0, ROWS)])
    pltpu.sync_copy(c_acc.at[pl.ds(r0, ROWS)],
                    counts_hbm.at[cid, pl.ds(r0, ROWS)])


@functools.cache
def _sc_segsum():
    mesh = plsc.VectorSubcoreMesh(
        core_axis_name="c", subcore_axis_name="s",
        num_cores=NC, num_subcores=NS)
    return pl.kernel(
        _sc_body,
        out_type=[
            jax.ShapeDtypeStruct((NC, ACC, NUM_FEATURES), jnp.float32),
            jax.ShapeDtypeStruct((NC, ACC, 16), jnp.float32),
            jax.ShapeDtypeStruct((BATCH,), jnp.int32),
        ],
        mesh=mesh,
        scratch_types=[
            pltpu.VMEM((KM, CHM), jnp.int32),
            pltpu.VMEM((1, CHM), jnp.int32),
            pltpu.VMEM((1, REM), jnp.int32),
            pltpu.VMEM((CHM, NUM_FEATURES), jnp.float32),
            pltpu.VMEM((CHM, NUM_FEATURES), jnp.float32),
            pltpu.VMEM((CHM, NUM_FEATURES), jnp.float32),
            pltpu.VMEM((CHM, NUM_FEATURES), jnp.float32),
            pltpu.VMEM((CHM, NUM_FEATURES), jnp.float32),
            pltpu.VMEM((CHM, 16), jnp.float32),
            pltpu.VMEM((TGT,), jnp.int32),
            pltpu.VMEM((TGT,), jnp.int32),
            pltpu.VMEM_SHARED((ACC, NUM_FEATURES), jnp.float32),
            pltpu.VMEM_SHARED((ACC, 16), jnp.float32),
        ] + [pltpu.SemaphoreType.DMA] * 13,
    )


def _tc_body(x_ref, s_ref, c_ref, t_ref, o_ref):
    s = s_ref[0] + s_ref[1]
    c2 = c_ref[0] + c_ref[1]
    sim = lax.dot_general(x_ref[...], s, (((1,), (1,)), ((), ())),
                          preferred_element_type=jnp.float32)
    w16 = jnp.full((1, 16), 1.0 / 16.0, jnp.float32)
    cnt = lax.dot_general(w16, c2, (((1,), (1,)), ((), ())),
                          preferred_element_type=jnp.float32)
    col = lax.broadcasted_iota(jnp.int32, (1, ACC), 1)
    mask = jnp.logical_and(cnt > 0.0, col < NUM_CLASSES)
    maskf = mask.astype(jnp.float32)
    simn = sim * (1.0 / TEMP) / jnp.where(cnt > 0.0, cnt, 1.0)
    e = jnp.exp(simn) * maskf
    denom = jnp.sum(e, axis=1, keepdims=True) + 1e-6
    colb = lax.broadcasted_iota(jnp.int32, (BATCH, ACC), 1)
    t = jnp.reshape(t_ref[...], (BATCH, 1))
    onehot = (colb == t).astype(jnp.float32)
    e_t = jnp.sum(e * onehot, axis=1, keepdims=True)
    p_t = e_t / denom
    log_pt = jnp.log(p_t + 1e-6)
    lb = -(1.0 + (1.0 - p_t) ** 4) * log_pt
    o_ref[0] = jnp.sum(lb) * (1.0 / BATCH)


_tc_loss = pl.pallas_call(
    _tc_body,
    out_shape=jax.ShapeDtypeStruct((1,), jnp.float32),
    out_specs=pl.BlockSpec(memory_space=pltpu.SMEM),
)


def kernel(inputs, another_inputs_full, indexes, features, labels):
    del another_inputs_full
    labels = labels.astype(jnp.int32)
    indexes = indexes.astype(jnp.int32)
    zs = jnp.zeros((ACC, NUM_FEATURES), jnp.float32)
    zc = jnp.zeros((ACC, 16), jnp.float32)
    ones = jnp.ones((CHM, 16), jnp.float32)
    sums, counts, targets = _sc_segsum()(
        features, labels, indexes, zs, zc, ones)
    loss = _tc_loss(inputs, sums, counts, targets)
    return loss[0]


# speedup vs baseline: 1.0068x; 1.0068x over previous
import functools

import jax
import jax.numpy as jnp
from jax import lax
from jax.experimental import pallas as pl
from jax.experimental.pallas import tpu as pltpu
from jax.experimental.pallas import tpu_sc as plsc

NUM_FEATURES = 128
NUM_SAMPLES = 100000
NUM_CLASSES = 1000
TEMP = 0.05
BATCH = 1024

NC = 2
NS = 16
NW = NC * NS
CHM = 128
KM = 24
MAIN = NW * KM * CHM
TAILF = 13
REM = 32
REM_OFF = MAIN + TAILF * CHM
ACC = 1024
ROWS = ACC // NS
TGT = BATCH // NW
NBUF = 4


def _sc_body(feat_hbm, labflat_hbm, idx_hbm, zs_hbm, zc_hbm, ones_hbm,
             sums_hbm, counts_hbm, targets_hbm,
             lab_v, labt_v, labr_v, feat_a, feat_b, feat_c, feat_d,
             featt_v, ones_v, idx_v, tgt_v, s_acc, c_acc,
             sem, lsem, tsem, sema, semb, semc, semd,
             ssema, ssemb, ssemc, ssemd, csem, tssem, isem, zsem):
    cid = lax.axis_index("c")
    sid = lax.axis_index("s")
    wid = cid * NS + sid

    bufs = (feat_a, feat_b, feat_c, feat_d)
    dsems = (sema, semb, semc, semd)
    ssems = (ssema, ssemb, ssemc, ssemd)

    def feat_src(k):
        base = pl.multiple_of((wid * KM + k) * CHM, 8)
        return feat_hbm.at[pl.ds(base, CHM)]

    def lab_src(k):
        base = pl.multiple_of((wid * KM + k) * CHM, 8)
        return labflat_hbm.at[pl.ds(base, CHM)]

    r0 = sid * ROWS
    tbase = pl.multiple_of(wid * TGT, 8)
    pltpu.async_copy(idx_hbm.at[pl.ds(tbase, TGT)], idx_v, isem)
    pltpu.async_copy(zs_hbm.at[pl.ds(r0, ROWS)], s_acc.at[pl.ds(r0, ROWS)],
                     zsem)
    pltpu.async_copy(zc_hbm.at[pl.ds(r0, ROWS)], c_acc.at[pl.ds(r0, ROWS)],
                     zsem)
    pltpu.async_copy(ones_hbm, ones_v, zsem)

    def lab_fire(k, carry):
        pltpu.async_copy(lab_src(k), lab_v.at[k], lsem)
        return carry

    lax.fori_loop(0, KM, lab_fire, 0)
    for r in range(NBUF):
        pltpu.async_copy(feat_src(r), bufs[r], dsems[r])

    tmain = pl.multiple_of(MAIN + wid * CHM, 8)

    @pl.when(wid < TAILF)
    def _():
        pltpu.async_copy(labflat_hbm.at[pl.ds(tmain, CHM)], labt_v.at[0],
                         lsem)
        pltpu.async_copy(feat_hbm.at[pl.ds(tmain, CHM)], featt_v, tsem)

    @pl.when(wid == TAILF)
    def _():
        pltpu.async_copy(labflat_hbm.at[pl.ds(REM_OFF, REM)], labr_v.at[0],
                         lsem)
        pltpu.async_copy(feat_hbm.at[pl.ds(REM_OFF, REM)],
                         featt_v.at[pl.ds(0, REM)], tsem)

    pltpu.make_async_copy(idx_hbm.at[pl.ds(tbase, TGT)], idx_v, isem).wait()
    pltpu.async_copy(labflat_hbm.at[idx_v], tgt_v, sem).wait()
    pltpu.async_copy(tgt_v, targets_hbm.at[pl.ds(tbase, TGT)], sem)

    pltpu.make_async_copy(zs_hbm.at[pl.ds(r0, ROWS)],
                          s_acc.at[pl.ds(r0, ROWS)], zsem).wait()
    pltpu.make_async_copy(zc_hbm.at[pl.ds(r0, ROWS)],
                          c_acc.at[pl.ds(r0, ROWS)], zsem).wait()
    pltpu.make_async_copy(ones_hbm, ones_v, zsem).wait()

    def lab_drain(k, carry):
        pltpu.make_async_copy(lab_src(k), lab_v.at[k], lsem).wait()
        return carry

    lax.fori_loop(0, KM, lab_drain, 0)

    @pl.when(wid < TAILF)
    def _():
        pltpu.make_async_copy(labflat_hbm.at[pl.ds(tmain, CHM)],
                              labt_v.at[0], lsem).wait()

    @pl.when(wid == TAILF)
    def _():
        pltpu.make_async_copy(labflat_hbm.at[pl.ds(REM_OFF, REM)],
                              labr_v.at[0], lsem).wait()

    plsc.subcore_barrier()

    @pl.when(wid < TAILF)
    def _():
        pltpu.make_async_copy(feat_hbm.at[pl.ds(tmain, CHM)], featt_v,
                              tsem).wait()
        pltpu.async_copy(featt_v, s_acc.at[labt_v.at[0]], tssem, add=True)
        pltpu.async_copy(ones_v, c_acc.at[labt_v.at[0]], csem, add=True)

    @pl.when(wid == TAILF)
    def _():
        pltpu.make_async_copy(feat_hbm.at[pl.ds(REM_OFF, REM)],
                              featt_v.at[pl.ds(0, REM)], tsem).wait()
        pltpu.async_copy(featt_v.at[pl.ds(0, REM)],
                         s_acc.at[labr_v.at[0]], tssem, add=True)
        pltpu.async_copy(ones_v.at[pl.ds(0, REM)],
                         c_acc.at[labr_v.at[0]], csem, add=True)

    def feat_scatter_start(k, r):
        pltpu.async_copy(bufs[r], s_acc.at[lab_v.at[k]], ssems[r], add=True)

    def feat_scatter_wait(k, r):
        pltpu.make_async_copy(bufs[r], s_acc.at[lab_v.at[k]],
                              ssems[r]).wait()

    def cnt_scatter_start(k):
        pltpu.async_copy(ones_v, c_acc.at[lab_v.at[k]], csem, add=True)

    def cnt_scatter_wait(k, carry):
        pltpu.make_async_copy(ones_v, c_acc.at[lab_v.at[k]], csem).wait()
        return carry

    def body(i, carry):
        k0 = i * NBUF
        for r in range(NBUF):
            pltpu.make_async_copy(feat_src(k0 + r), bufs[r],
                                  dsems[r]).wait()
            feat_scatter_start(k0 + r, r)
            cnt_scatter_start(k0 + r)
        for r in range(NBUF):
            @pl.when(k0 + NBUF + r < KM)
            def _():
                feat_scatter_wait(k0 + r, r)
                pltpu.async_copy(feat_src(k0 + NBUF + r), bufs[r],
                                 dsems[r])
        return carry

    lax.fori_loop(0, KM // NBUF, body, 0)

    for r in range(NBUF):
        feat_scatter_wait(KM - NBUF + r, r)

    @pl.when(wid < TAILF)
    def _():
        pltpu.make_async_copy(featt_v, s_acc.at[labt_v.at[0]],
                              tssem).wait()
        pltpu.make_async_copy(ones_v, c_acc.at[labt_v.at[0]], csem).wait()

    @pl.when(wid == TAILF)
    def _():
        pltpu.make_async_copy(featt_v.at[pl.ds(0, REM)],
                              s_acc.at[labr_v.at[0]], tssem).wait()
        pltpu.make_async_copy(ones_v.at[pl.ds(0, REM)],
                              c_acc.at[labr_v.at[0]], csem).wait()

    lax.fori_loop(0, KM, cnt_scatter_wait, 0)

    plsc.subcore_barrier()

    pltpu.async_copy(s_acc.at[pl.ds(r0, ROWS)],
                     sums_hbm.at[cid, pl.ds(r0, ROWS)], zsem)
    pltpu.async_copy(c_acc.at[pl.ds(r0, ROWS)],
                     counts_hbm.at[cid, pl.ds(r0, ROWS)], zsem)
    pltpu.make_async_copy(s_acc.at[pl.ds(r0, ROWS)],
                          sums_hbm.at[cid, pl.ds(r0, ROWS)], zsem).wait()
    pltpu.make_async_copy(c_acc.at[pl.ds(r0, ROWS)],
                          counts_hbm.at[cid, pl.ds(r0, ROWS)], zsem).wait()
    pltpu.make_async_copy(tgt_v, targets_hbm.at[pl.ds(tbase, TGT)],
                          sem).wait()


@functools.cache
def _sc_segsum():
    mesh = plsc.VectorSubcoreMesh(
        core_axis_name="c", subcore_axis_name="s",
        num_cores=NC, num_subcores=NS)
    return pl.kernel(
        _sc_body,
        out_type=[
            jax.ShapeDtypeStruct((NC, ACC, NUM_FEATURES), jnp.float32),
            jax.ShapeDtypeStruct((NC, ACC, 16), jnp.float32),
            jax.ShapeDtypeStruct((BATCH,), jnp.int32),
        ],
        mesh=mesh,
        scratch_types=[
            pltpu.VMEM((KM, CHM), jnp.int32),
            pltpu.VMEM((1, CHM), jnp.int32),
            pltpu.VMEM((1, REM), jnp.int32),
            pltpu.VMEM((CHM, NUM_FEATURES), jnp.float32),
            pltpu.VMEM((CHM, NUM_FEATURES), jnp.float32),
            pltpu.VMEM((CHM, NUM_FEATURES), jnp.float32),
            pltpu.VMEM((CHM, NUM_FEATURES), jnp.float32),
            pltpu.VMEM((CHM, NUM_FEATURES), jnp.float32),
            pltpu.VMEM((CHM, 16), jnp.float32),
            pltpu.VMEM((TGT,), jnp.int32),
            pltpu.VMEM((TGT,), jnp.int32),
            pltpu.VMEM_SHARED((ACC, NUM_FEATURES), jnp.float32),
            pltpu.VMEM_SHARED((ACC, 16), jnp.float32),
        ] + [pltpu.SemaphoreType.DMA] * 15,
    )


def _tc_body(x_ref, s_ref, c_ref, t_ref, o_ref):
    s = s_ref[0] + s_ref[1]
    c2 = c_ref[0] + c_ref[1]
    sim = lax.dot_general(x_ref[...], s, (((1,), (1,)), ((), ())),
                          preferred_element_type=jnp.float32)
    w16 = jnp.full((1, 16), 1.0 / 16.0, jnp.float32)
    cnt = lax.dot_general(w16, c2, (((1,), (1,)), ((), ())),
                          preferred_element_type=jnp.float32)
    col = lax.broadcasted_iota(jnp.int32, (1, ACC), 1)
    mask = jnp.logical_and(cnt > 0.0, col < NUM_CLASSES)
    maskf = mask.astype(jnp.float32)
    simn = sim * (1.0 / TEMP) / jnp.where(cnt > 0.0, cnt, 1.0)
    e = jnp.exp(simn) * maskf
    denom = jnp.sum(e, axis=1, keepdims=True) + 1e-6
    colb = lax.broadcasted_iota(jnp.int32, (BATCH, ACC), 1)
    t = jnp.reshape(t_ref[...], (BATCH, 1))
    onehot = (colb == t).astype(jnp.float32)
    e_t = jnp.sum(e * onehot, axis=1, keepdims=True)
    p_t = e_t / denom
    log_pt = jnp.log(p_t + 1e-6)
    lb = -(1.0 + (1.0 - p_t) ** 4) * log_pt
    o_ref[0] = jnp.sum(lb) * (1.0 / BATCH)


_tc_loss = pl.pallas_call(
    _tc_body,
    out_shape=jax.ShapeDtypeStruct((1,), jnp.float32),
    out_specs=pl.BlockSpec(memory_space=pltpu.SMEM),
)


def kernel(inputs, another_inputs_full, indexes, features, labels):
    del another_inputs_full
    labels = labels.astype(jnp.int32)
    indexes = indexes.astype(jnp.int32)
    zs = jnp.zeros((ACC, NUM_FEATURES), jnp.float32)
    zc = jnp.zeros((ACC, 16), jnp.float32)
    ones = jnp.ones((CHM, 16), jnp.float32)
    sums, counts, targets = _sc_segsum()(
        features, labels, indexes, zs, zc, ones)
    loss = _tc_loss(inputs, sums, counts, targets)
    return loss[0]
